# Initial kernel scaffold; baseline (speedup 1.0000x reference)
import jax
import jax.numpy as jnp
from jax.experimental import pallas as pl


def kernel(inputs, emb_table, w1_table, W0, b0, W1, b1, W2, b2):
    raise NotImplementedError("write your pallas kernel here")


# speedup vs baseline: 1.2999x; 1.2999x over previous
import functools

import jax
import jax.numpy as jnp
from jax import lax
from jax.experimental import pallas as pl
from jax.experimental.pallas import tpu as pltpu
from jax.experimental.pallas import tpu_sc as plsc

_F = 26
_D = 16
_CHUNK = 128
_NW = 32
_L = 16


def _sc_fm(x_chunks, emb_table, w1_flat, batch):
    num_chunks = batch // _CHUNK
    chunks_per_w = num_chunks // _NW

    mesh = plsc.VectorSubcoreMesh(core_axis_name="c", subcore_axis_name="s")

    @functools.partial(
        pl.kernel,
        out_type=jax.ShapeDtypeStruct((num_chunks, _CHUNK, _D), jnp.float32),
        mesh=mesh,
        scratch_types=[
            pltpu.VMEM((_F, _CHUNK), jnp.int32),
            pltpu.VMEM((_F * _CHUNK, _D), jnp.float32),
            pltpu.VMEM((_F, _CHUNK), jnp.float32),
            pltpu.VMEM((_CHUNK + _L, ), jnp.float32),
            pltpu.VMEM((_CHUNK, _D), jnp.float32),
            pltpu.SemaphoreType.DMA,
        ],
        compiler_params=pltpu.CompilerParams(use_tc_tiling_on_sc=False),
    )
    def fm_kernel(x_hbm, emb_hbm, w1_hbm, out_hbm,
                  idx_v, rows_v, w1_v, lin_v, h_v, sem):
        wid = lax.axis_index("s") * 2 + lax.axis_index("c")

        for c in range(chunks_per_w):
            g = wid * chunks_per_w + c
            pltpu.sync_copy(x_hbm.at[g], idx_v)

            descs = []
            for f in range(_F):
                descs.append(pltpu.async_copy(
                    emb_hbm.at[idx_v.at[f]],
                    rows_v.at[pl.ds(f * _CHUNK, _CHUNK), :],
                    sem,
                ))
                descs.append(pltpu.async_copy(
                    w1_hbm.at[idx_v.at[f]],
                    w1_v.at[f],
                    sem,
                ))
            for dsc in descs:
                dsc.wait()

            for k in range(_CHUNK // _L):
                acc = w1_v[0, pl.ds(k * _L, _L)]
                for f in range(1, _F):
                    acc = acc + w1_v[f, pl.ds(k * _L, _L)]
                lin_v[pl.ds(k * _L, _L)] = acc

            def body(b, carry):
                v = rows_v[b]
                s = v
                sq = v * v
                for f in range(1, _F):
                    v = rows_v[f * _CHUNK + b]
                    s = s + v
                    sq = sq + v * v
                lin = lin_v[pl.ds(b, _L)][0]
                h_v[b] = 0.5 * (s * s - sq) + lin
                return carry

            lax.fori_loop(0, _CHUNK, body, 0, unroll=False)
            pltpu.sync_copy(h_v, out_hbm.at[g])

    return fm_kernel(x_chunks, emb_table, w1_flat)


def _tc_mlp(h, W0, b0, W1, b1, W2, b2):
    batch = h.shape[0]

    def mlp_kernel(h_ref, w0_ref, b0_ref, w1_ref, b1_ref, w2_ref, b2_ref, o_ref):
        z = h_ref[...]
        z = jnp.maximum(
            jnp.dot(z, w0_ref[...], preferred_element_type=jnp.float32)
            + b0_ref[...], 0.0)
        z = jnp.maximum(
            jnp.dot(z, w1_ref[...], preferred_element_type=jnp.float32)
            + b1_ref[...], 0.0)
        out = jnp.sum(z * w2_ref[...], axis=1) + b2_ref[0, 0]
        o_ref[...] = jax.nn.sigmoid(out)

    return pl.pallas_call(
        mlp_kernel,
        out_shape=jax.ShapeDtypeStruct((batch,), jnp.float32),
    )(h, W0, b0.reshape(1, -1), W1, b1.reshape(1, -1), W2.reshape(1, -1),
      b2.reshape(1, 1))


def kernel(inputs, emb_table, w1_table, W0, b0, W1, b1, W2, b2):
    batch, ncols = inputs.shape
    half = ncols // 2
    x = inputs[:, half:]
    x_chunks = x.reshape(batch // _CHUNK, _CHUNK, _F).transpose(0, 2, 1)
    h = _sc_fm(x_chunks, emb_table, w1_table.reshape(-1), batch)
    return _tc_mlp(h.reshape(batch, _D), W0, b0, W1, b1, W2, b2)
